# baseline (speedup 1.0000x reference)
import functools

import jax
import jax.numpy as jnp
from jax import lax
from jax.experimental import pallas as pl
from jax.experimental.pallas import tpu as pltpu
from jax.experimental.pallas import tpu_sc as plsc

H_DIM = 64
GRID = 8
NS = 2.0
PED = 64
TG = GRID * GRID
POOL_IN = TG * H_DIM
SB = 4
SBP = SB * PED
SPILL_K = 1024
SC_LANES = 16
SCB = 2


def _sc_scene_body(ox_ref, oy_ref, axs_ref, ays_ref, gp_ref):
    jiota = lax.broadcasted_iota(jnp.int32, (SC_LANES,), 0)

    @plsc.parallel_loop(0, SCB * PED, unroll=4)
    def _(r):
        ax = axs_ref[r, :]
        ay = ays_ref[r, :]
        tlx = ax - NS / 2.0
        tly = ay + NS / 2.0
        brx = ax + NS / 2.0
        bry = ay - NS / 2.0
        i = lax.rem(r, PED)
        sc = lax.div(r, PED)
        for jc in range(PED // SC_LANES):
            ox = ox_ref[sc, pl.ds(jc * SC_LANES, SC_LANES)]
            oy = oy_ref[sc, pl.ds(jc * SC_LANES, SC_LANES)]
            fx = (ox - tlx) / NS * GRID
            fy = (tly - oy) / NS * GRID
            cx = fx.astype(jnp.int32)
            cy = fy.astype(jnp.int32)
            x_bound = (ox >= brx) | (ox <= tlx)
            y_bound = (oy >= tly) | (oy <= bry)
            eye = (jiota + (jc * SC_LANES)) == i
            within = x_bound | y_bound | eye
            gp_ref[r, pl.ds(jc * SC_LANES, SC_LANES)] = jnp.where(
                within, -1, cx + cy * GRID)


def _sc_binning(oxa, oya, axs, ays):
    num_seqs = oxa.shape[0]
    mesh = plsc.VectorSubcoreMesh(core_axis_name="c", subcore_axis_name="s")

    @pl.kernel(
        out_type=jax.ShapeDtypeStruct((num_seqs * PED, PED), jnp.int32),
        mesh=mesh)
    def sc_kernel(ox_hbm, oy_hbm, axs_hbm, ays_hbm, gp_hbm):
        pltpu.emit_pipeline(
            _sc_scene_body,
            grid=(num_seqs // SCB,),
            in_specs=[
                pl.BlockSpec((SCB, PED), lambda s: (s, 0)),
                pl.BlockSpec((SCB, PED), lambda s: (s, 0)),
                pl.BlockSpec((SCB * PED, SC_LANES), lambda s: (s, 0)),
                pl.BlockSpec((SCB * PED, SC_LANES), lambda s: (s, 0)),
            ],
            out_specs=[pl.BlockSpec((SCB * PED, PED), lambda s: (s, 0))],
            core_axis_name=("c", "s"),
            dimension_semantics=(pltpu.PARALLEL,),
        )(ox_hbm, oy_hbm, axs_hbm, ays_hbm, gp_hbm)

    return sc_kernel(oxa, oya, axs, ays)


def _scene_kernel(h_ref, gp_ref, w1_ref, kc_ref, y_ref, stats_ref):
    step = pl.program_id(0)

    zb = jax.lax.dot(h_ref[...], w1_ref[...],
                     preferred_element_type=jnp.float32
                     ).astype(jnp.bfloat16)

    gp = gp_ref[...]

    first_row = (lax.broadcasted_iota(jnp.int32, (SBP, PED), 0) % PED) == 0
    shifted = jnp.concatenate(
        [jnp.full((1, PED), -128, jnp.int32), gp[:SBP - 1, :] - 64], axis=0)
    spill = jnp.where(first_row, -128, shifted)

    kc = jnp.broadcast_to(kc_ref[0:1, :], (SBP, TG * PED))
    gpt = jnp.tile(gp, (1, TG))
    spt = jnp.tile(spill, (1, SPILL_K // PED))
    m_main = (gpt == kc).astype(jnp.bfloat16)
    m_low = (m_main[:, :SPILL_K]
             + (spt == kc[:, :SPILL_K]).astype(jnp.bfloat16))

    for s in range(SB):
        zs = zb[s * PED:(s + 1) * PED, :]
        zf = jnp.concatenate(
            [zs[:, c * H_DIM:(c + 1) * H_DIM] for c in range(TG)],
            axis=0)

        y = (jax.lax.dot(m_low[s * PED:(s + 1) * PED, :], zf[:SPILL_K, :],
                         preferred_element_type=jnp.float32)
             + jax.lax.dot(m_main[s * PED:(s + 1) * PED, SPILL_K:],
                           zf[SPILL_K:, :],
                           preferred_element_type=jnp.float32))
        y_ref[s * PED:(s + 1) * PED, :] = y

        @pl.when((step == 0) & (s == 0))
        def _():
            stats_ref[...] = jnp.zeros_like(stats_ref)

        stats_ref[0:1, :] += jnp.sum(y, axis=0, keepdims=True)
        stats_ref[1:2, :] += jnp.sum(y * y, axis=0, keepdims=True)


def _norm_kernel(y_ref, stats_ref, gamma_ref, beta_ref, out_ref, *, n_rows,
                 nch):
    ssum = stats_ref[0:1, :]
    ssq = stats_ref[1:2, :]
    for k in range(1, nch):
        ssum = ssum + stats_ref[8 * k:8 * k + 1, :]
        ssq = ssq + stats_ref[8 * k + 1:8 * k + 2, :]
    mu = ssum * (1.0 / n_rows)
    ex2 = ssq * (1.0 / n_rows)
    var = ex2 - mu * mu
    inv = 1.0 / jnp.sqrt(var + 1e-5)
    y = y_ref[...]
    out = (y - mu) * (inv * gamma_ref[0:1, :]) + beta_ref[0:1, :]
    out_ref[...] = jnp.maximum(out, 0.0)


def kernel(h_states, seq_start_end, end_pos, W1, b1, gamma, beta):
    n = h_states.shape[0]
    num_seqs = seq_start_end.shape[0]
    assert n == num_seqs * PED

    oxa = end_pos[:, 0].reshape(num_seqs, PED)
    oya = end_pos[:, 1].reshape(num_seqs, PED)
    w1p = W1.reshape(TG, H_DIM, H_DIM).transpose(1, 0, 2).reshape(
        H_DIM, POOL_IN).astype(jnp.bfloat16)
    hb = h_states.astype(jnp.bfloat16)
    kc8 = jnp.broadcast_to(
        (jnp.arange(TG * PED, dtype=jnp.int32) // PED).reshape(1, TG * PED),
        (8, TG * PED))
    gammar = jnp.broadcast_to(gamma.reshape(1, H_DIM), (8, H_DIM))
    betar = jnp.broadcast_to(beta.reshape(1, H_DIM), (8, H_DIM))
    axs = jnp.broadcast_to(end_pos[:, 0].reshape(n, 1), (n, SC_LANES))
    ays = jnp.broadcast_to(end_pos[:, 1].reshape(n, 1), (n, SC_LANES))

    NCH = 1
    cs = num_seqs // NCH
    cn = cs * PED
    ys, stats_list = [], []
    for k in range(NCH):
        gp_k = _sc_binning(oxa[k * cs:(k + 1) * cs],
                           oya[k * cs:(k + 1) * cs],
                           axs[k * cn:(k + 1) * cn],
                           ays[k * cn:(k + 1) * cn])
        y_k, st_k = pl.pallas_call(
            _scene_kernel,
            grid=(cs // SB,),
            in_specs=[
                pl.BlockSpec((SBP, H_DIM), lambda t: (t, 0)),
                pl.BlockSpec((SBP, PED), lambda t: (t, 0)),
                pl.BlockSpec((H_DIM, POOL_IN), lambda t: (0, 0)),
                pl.BlockSpec((8, TG * PED), lambda t: (0, 0)),
            ],
            out_specs=[
                pl.BlockSpec((SBP, H_DIM), lambda t: (t, 0)),
                pl.BlockSpec((8, H_DIM), lambda t: (0, 0)),
            ],
            out_shape=[
                jax.ShapeDtypeStruct((cn, H_DIM), jnp.float32),
                jax.ShapeDtypeStruct((8, H_DIM), jnp.float32),
            ],
        )(hb[k * cn:(k + 1) * cn], gp_k, w1p, kc8)
        ys.append(y_k)
        stats_list.append(st_k)

    stats = jnp.concatenate(stats_list, axis=0)

    rows_blk = 2048
    outs = []
    for k in range(NCH):
        outs.append(pl.pallas_call(
            functools.partial(_norm_kernel, n_rows=n, nch=NCH),
            grid=(cn // rows_blk,),
            in_specs=[
                pl.BlockSpec((rows_blk, H_DIM), lambda r: (r, 0)),
                pl.BlockSpec((NCH * 8, H_DIM), lambda r: (0, 0)),
                pl.BlockSpec((8, H_DIM), lambda r: (0, 0)),
                pl.BlockSpec((8, H_DIM), lambda r: (0, 0)),
            ],
            out_specs=pl.BlockSpec((rows_blk, H_DIM), lambda r: (r, 0)),
            out_shape=jax.ShapeDtypeStruct((cn, H_DIM), jnp.float32),
        )(ys[k], stats, gammar, betar))
    return jnp.concatenate(outs, axis=0)


# speedup vs baseline: 1.0211x; 1.0211x over previous
import functools

import jax
import jax.numpy as jnp
from jax import lax
from jax.experimental import pallas as pl
from jax.experimental.pallas import tpu as pltpu
from jax.experimental.pallas import tpu_sc as plsc

H_DIM = 64
GRID = 8
NS = 2.0
PED = 64
TG = GRID * GRID
POOL_IN = TG * H_DIM
SB = 4
SBP = SB * PED
SPILL_K = 1024
SC_LANES = 16


def _sc_scene_body(ox_ref, oy_ref, axs_ref, ays_ref, gp_ref):
    jiota = lax.broadcasted_iota(jnp.int32, (SC_LANES,), 0)

    @plsc.parallel_loop(0, PED, unroll=4)
    def _(i):
        ax = axs_ref[i, :]
        ay = ays_ref[i, :]
        tlx = ax - NS / 2.0
        tly = ay + NS / 2.0
        brx = ax + NS / 2.0
        bry = ay - NS / 2.0
        for jc in range(PED // SC_LANES):
            ox = ox_ref[0, pl.ds(jc * SC_LANES, SC_LANES)]
            oy = oy_ref[0, pl.ds(jc * SC_LANES, SC_LANES)]
            fx = (ox - tlx) / NS * GRID
            fy = (tly - oy) / NS * GRID
            cx = fx.astype(jnp.int32)
            cy = fy.astype(jnp.int32)
            x_bound = (ox >= brx) | (ox <= tlx)
            y_bound = (oy >= tly) | (oy <= bry)
            eye = (jiota + (jc * SC_LANES)) == i
            within = x_bound | y_bound | eye
            gp_ref[i, pl.ds(jc * SC_LANES, SC_LANES)] = jnp.where(
                within, -1, cx + cy * GRID)


def _sc_binning(oxa, oya, axs, ays):
    num_seqs = oxa.shape[0]
    mesh = plsc.VectorSubcoreMesh(core_axis_name="c", subcore_axis_name="s")

    @pl.kernel(
        out_type=jax.ShapeDtypeStruct((num_seqs * PED, PED), jnp.int32),
        mesh=mesh)
    def sc_kernel(ox_hbm, oy_hbm, axs_hbm, ays_hbm, gp_hbm):
        pltpu.emit_pipeline(
            _sc_scene_body,
            grid=(num_seqs,),
            in_specs=[
                pl.BlockSpec((1, PED), lambda s: (s, 0)),
                pl.BlockSpec((1, PED), lambda s: (s, 0)),
                pl.BlockSpec((PED, SC_LANES), lambda s: (s, 0)),
                pl.BlockSpec((PED, SC_LANES), lambda s: (s, 0)),
            ],
            out_specs=[pl.BlockSpec((PED, PED), lambda s: (s, 0))],
            core_axis_name=("c", "s"),
            dimension_semantics=(pltpu.PARALLEL,),
        )(ox_hbm, oy_hbm, axs_hbm, ays_hbm, gp_hbm)

    return sc_kernel(oxa, oya, axs, ays)


def _scene_kernel(h_ref, gp_ref, w1_ref, kc_ref, y_ref, stats_ref):
    step = pl.program_id(0)

    zb = jax.lax.dot(h_ref[...], w1_ref[...],
                     preferred_element_type=jnp.float32
                     ).astype(jnp.bfloat16)

    gp = gp_ref[...]

    first_row = (lax.broadcasted_iota(jnp.int32, (SBP, PED), 0) % PED) == 0
    shifted = jnp.concatenate(
        [jnp.full((1, PED), -128, jnp.int32), gp[:SBP - 1, :] - 64], axis=0)
    spill = jnp.where(first_row, -128, shifted)

    kc = jnp.broadcast_to(kc_ref[0:1, :], (SBP, TG * PED))
    gpt = jnp.tile(gp, (1, TG))
    spt = jnp.tile(spill, (1, SPILL_K // PED))
    m_main = (gpt == kc).astype(jnp.bfloat16)
    m_low = (m_main[:, :SPILL_K]
             + (spt == kc[:, :SPILL_K]).astype(jnp.bfloat16))

    for s in range(SB):
        zs = zb[s * PED:(s + 1) * PED, :]
        zf = jnp.concatenate(
            [zs[:, c * H_DIM:(c + 1) * H_DIM] for c in range(TG)],
            axis=0)

        y = (jax.lax.dot(m_low[s * PED:(s + 1) * PED, :], zf[:SPILL_K, :],
                         preferred_element_type=jnp.float32)
             + jax.lax.dot(m_main[s * PED:(s + 1) * PED, SPILL_K:],
                           zf[SPILL_K:, :],
                           preferred_element_type=jnp.float32))
        y_ref[s * PED:(s + 1) * PED, :] = y

        @pl.when((step == 0) & (s == 0))
        def _():
            stats_ref[...] = jnp.zeros_like(stats_ref)

        stats_ref[0:1, :] += jnp.sum(y, axis=0, keepdims=True)
        stats_ref[1:2, :] += jnp.sum(y * y, axis=0, keepdims=True)


def _norm_kernel(y_ref, stats_ref, gamma_ref, beta_ref, out_ref, *, n_rows,
                 nch):
    ssum = stats_ref[0:1, :]
    ssq = stats_ref[1:2, :]
    for k in range(1, nch):
        ssum = ssum + stats_ref[8 * k:8 * k + 1, :]
        ssq = ssq + stats_ref[8 * k + 1:8 * k + 2, :]
    mu = ssum * (1.0 / n_rows)
    ex2 = ssq * (1.0 / n_rows)
    var = ex2 - mu * mu
    inv = 1.0 / jnp.sqrt(var + 1e-5)
    y = y_ref[...]
    out = (y - mu) * (inv * gamma_ref[0:1, :]) + beta_ref[0:1, :]
    out_ref[...] = jnp.maximum(out, 0.0)


def kernel(h_states, seq_start_end, end_pos, W1, b1, gamma, beta):
    n = h_states.shape[0]
    num_seqs = seq_start_end.shape[0]
    assert n == num_seqs * PED

    oxa = end_pos[:, 0].reshape(num_seqs, PED)
    oya = end_pos[:, 1].reshape(num_seqs, PED)
    w1p = W1.reshape(TG, H_DIM, H_DIM).transpose(1, 0, 2).reshape(
        H_DIM, POOL_IN).astype(jnp.bfloat16)
    hb = h_states.astype(jnp.bfloat16)
    kc8 = jnp.broadcast_to(
        (jnp.arange(TG * PED, dtype=jnp.int32) // PED).reshape(1, TG * PED),
        (8, TG * PED))
    gammar = jnp.broadcast_to(gamma.reshape(1, H_DIM), (8, H_DIM))
    betar = jnp.broadcast_to(beta.reshape(1, H_DIM), (8, H_DIM))
    axs = jnp.broadcast_to(end_pos[:, 0].reshape(n, 1), (n, SC_LANES))
    ays = jnp.broadcast_to(end_pos[:, 1].reshape(n, 1), (n, SC_LANES))

    NCH = 1
    cs = num_seqs // NCH
    cn = cs * PED
    ys, stats_list = [], []
    for k in range(NCH):
        gp_k = _sc_binning(oxa[k * cs:(k + 1) * cs],
                           oya[k * cs:(k + 1) * cs],
                           axs[k * cn:(k + 1) * cn],
                           ays[k * cn:(k + 1) * cn])
        y_k, st_k = pl.pallas_call(
            _scene_kernel,
            grid=(cs // SB,),
            in_specs=[
                pl.BlockSpec((SBP, H_DIM), lambda t: (t, 0)),
                pl.BlockSpec((SBP, PED), lambda t: (t, 0)),
                pl.BlockSpec((H_DIM, POOL_IN), lambda t: (0, 0)),
                pl.BlockSpec((8, TG * PED), lambda t: (0, 0)),
            ],
            out_specs=[
                pl.BlockSpec((SBP, H_DIM), lambda t: (t, 0)),
                pl.BlockSpec((8, H_DIM), lambda t: (0, 0)),
            ],
            out_shape=[
                jax.ShapeDtypeStruct((cn, H_DIM), jnp.float32),
                jax.ShapeDtypeStruct((8, H_DIM), jnp.float32),
            ],
        )(hb[k * cn:(k + 1) * cn], gp_k, w1p, kc8)
        ys.append(y_k)
        stats_list.append(st_k)

    stats = jnp.concatenate(stats_list, axis=0)

    rows_blk = 2048
    outs = []
    for k in range(NCH):
        outs.append(pl.pallas_call(
            functools.partial(_norm_kernel, n_rows=n, nch=NCH),
            grid=(cn // rows_blk,),
            in_specs=[
                pl.BlockSpec((rows_blk, H_DIM), lambda r: (r, 0)),
                pl.BlockSpec((NCH * 8, H_DIM), lambda r: (0, 0)),
                pl.BlockSpec((8, H_DIM), lambda r: (0, 0)),
                pl.BlockSpec((8, H_DIM), lambda r: (0, 0)),
            ],
            out_specs=pl.BlockSpec((rows_blk, H_DIM), lambda r: (r, 0)),
            out_shape=jax.ShapeDtypeStruct((cn, H_DIM), jnp.float32),
        )(ys[k], stats, gammar, betar))
    return jnp.concatenate(outs, axis=0)
